# baseline (speedup 1.0000x reference)
import functools

import jax
import jax.numpy as jnp
from jax import lax
from jax.experimental import pallas as pl
from jax.experimental.pallas import tpu as pltpu
from jax.experimental.pallas import tpu_sc as plsc

DICT_NUM = 10
DICT_DIM = 80
COMMON = 96

_NC = 2
_NS = 16
_NW = _NC * _NS
_CHUNK = 64


def _sc_gather(emb, idx):
    vocab, d = emb.shape
    b = idx.shape[0]
    b_per_w = b // _NW
    mesh = plsc.VectorSubcoreMesh(core_axis_name="c", subcore_axis_name="s")

    @functools.partial(
        pl.kernel,
        mesh=mesh,
        out_type=jax.ShapeDtypeStruct((b, d), jnp.float32),
        scratch_types=[
            pltpu.VMEM((b_per_w,), jnp.int32),
            pltpu.VMEM((_CHUNK, d), jnp.float32),
            pltpu.SemaphoreType.DMA,
        ],
        compiler_params=pltpu.CompilerParams(use_tc_tiling_on_sc=False),
    )
    def gather_kernel(table_hbm, idx_hbm, out_hbm, idx_v, rows_v, sem):
        wid = lax.axis_index("s") * _NC + lax.axis_index("c")
        base = wid * b_per_w
        pltpu.sync_copy(idx_hbm.at[pl.ds(base, b_per_w)], idx_v)

        @pl.loop(0, b_per_w, step=_CHUNK)
        def _(c):
            pltpu.async_copy(
                table_hbm.at[idx_v.at[pl.ds(c, _CHUNK)]], rows_v, sem
            ).wait()
            pltpu.sync_copy(rows_v, out_hbm.at[pl.ds(base + c, _CHUNK)])

    return gather_kernel(emb, idx)


def _lrelu(x):
    return jnp.where(x >= 0, x, 0.01 * x)


def _dense_body(fc_ref, fl_ref, ve_ref, w1_ref, b1_ref, w2_ref, b2_ref,
                w3_ref, b3_ref, wloc_ref, bloc_ref, wcls_ref,
                cls_ref, cos_ref, vcomp_ref, vlc_ref, vlcm_ref):
    f32 = jnp.float32
    x = fc_ref[...]
    h = _lrelu(jnp.dot(x, w1_ref[...], preferred_element_type=f32) + b1_ref[...])
    h = _lrelu(jnp.dot(h, w2_ref[...], preferred_element_type=f32) + b2_ref[...])
    v_comp = _lrelu(jnp.dot(h, w3_ref[...], preferred_element_type=f32) + b3_ref[...])
    v_loc = _lrelu(jnp.dot(fl_ref[...], wloc_ref[...], preferred_element_type=f32)
                   + bloc_ref[...])

    ve = ve_ref[...]
    vc16 = v_comp[:, :16]
    vc80 = v_comp[:, 16:]
    n1 = jnp.sqrt(jnp.sum(v_comp * v_comp, axis=1, keepdims=True))
    nloc2 = jnp.sum(v_loc * v_loc, axis=1, keepdims=True)
    dloc = jnp.sum(vc16 * v_loc, axis=1, keepdims=True)

    best = None
    chosen = None
    pieces = []
    for k in range(DICT_NUM):
        ve_k = ve[:, k * DICT_DIM:(k + 1) * DICT_DIM]
        pieces.append(v_loc)
        pieces.append(ve_k)
        dot_k = dloc + jnp.sum(vc80 * ve_k, axis=1, keepdims=True)
        n2_k = jnp.sqrt(nloc2 + jnp.sum(ve_k * ve_k, axis=1, keepdims=True))
        cos_k = dot_k / jnp.maximum(n1 * n2_k, 1e-8)
        if best is None:
            best = cos_k
            chosen = ve_k
        else:
            upd = cos_k > best
            best = jnp.where(upd, cos_k, best)
            chosen = jnp.where(upd, ve_k, chosen)

    vlc_ref[...] = jnp.concatenate(pieces, axis=1)
    cos_ref[...] = best
    vcomp_ref[...] = v_comp
    vlcm = jnp.concatenate([v_loc, chosen], axis=1)
    vlcm_ref[...] = vlcm
    v_diff = jnp.abs(vlcm - v_comp)
    cls_ref[...] = jnp.dot(v_diff, wcls_ref[...], preferred_element_type=f32)


def _dense(feat_comp, feat_loc, v_emb, W1, b1, W2, b2, W3, b3, Wloc, bloc, Wcls):
    b = feat_comp.shape[0]
    bb = 1024
    grid = (b // bb,)
    f32 = jnp.float32

    def row_spec(cols):
        return pl.BlockSpec((bb, cols), lambda i: (i, 0))

    def full_spec(shape):
        return pl.BlockSpec(shape, lambda i: (0, 0))

    weights = [W1.T, b1.reshape(1, -1), W2.T, b2.reshape(1, -1),
               W3.T, b3.reshape(1, -1), Wloc.T, bloc.reshape(1, -1), Wcls.T]

    out = pl.pallas_call(
        _dense_body,
        grid=grid,
        in_specs=[
            row_spec(feat_comp.shape[1]),
            row_spec(feat_loc.shape[1]),
            row_spec(v_emb.shape[1]),
        ] + [full_spec(w.shape) for w in weights],
        out_specs=[
            row_spec(2),
            row_spec(1),
            row_spec(COMMON),
            row_spec(DICT_NUM * COMMON),
            row_spec(COMMON),
        ],
        out_shape=[
            jax.ShapeDtypeStruct((b, 2), f32),
            jax.ShapeDtypeStruct((b, 1), f32),
            jax.ShapeDtypeStruct((b, COMMON), f32),
            jax.ShapeDtypeStruct((b, DICT_NUM * COMMON), f32),
            jax.ShapeDtypeStruct((b, COMMON), f32),
        ],
    )(feat_comp, feat_loc, v_emb, *weights)
    return out


@jax.jit
def kernel(feat_comp, feat_loc, id_loc, W1, b1, W2, b2, W3, b3, emb, Wloc, bloc, Wcls):
    v_emb = _sc_gather(emb, id_loc.astype(jnp.int32))
    cls, cos, vcomp, vlc, vlcm = _dense(
        feat_comp, feat_loc, v_emb, W1, b1, W2, b2, W3, b3, Wloc, bloc, Wcls)
    b = feat_comp.shape[0]
    return (cls, cos, vcomp, vlc.reshape(b, DICT_NUM, COMMON), vlcm)


# speedup vs baseline: 1.1271x; 1.1271x over previous
import functools

import jax
import jax.numpy as jnp
from jax import lax
from jax.experimental import pallas as pl
from jax.experimental.pallas import tpu as pltpu
from jax.experimental.pallas import tpu_sc as plsc

DICT_NUM = 10
DICT_DIM = 80
COMMON = 96

_NC = 2
_NS = 16
_NW = _NC * _NS
_CHUNK = 64


def _sc_gather(emb, idx):
    vocab, d = emb.shape
    b = idx.shape[0]
    b_per_w = b // _NW
    mesh = plsc.VectorSubcoreMesh(core_axis_name="c", subcore_axis_name="s")

    @functools.partial(
        pl.kernel,
        mesh=mesh,
        out_type=jax.ShapeDtypeStruct((b, d), jnp.float32),
        scratch_types=[
            pltpu.VMEM((b_per_w,), jnp.int32),
            pltpu.VMEM((_CHUNK, d), jnp.float32),
            pltpu.SemaphoreType.DMA,
        ],
    )
    def gather_kernel(table_hbm, idx_hbm, out_hbm, idx_v, rows_v, sem):
        wid = lax.axis_index("s") * _NC + lax.axis_index("c")
        base = wid * b_per_w
        pltpu.sync_copy(idx_hbm.at[pl.ds(base, b_per_w)], idx_v)

        @pl.loop(0, b_per_w, step=_CHUNK)
        def _(c):
            pltpu.async_copy(
                table_hbm.at[idx_v.at[pl.ds(c, _CHUNK)]], rows_v, sem
            ).wait()
            pltpu.sync_copy(rows_v, out_hbm.at[pl.ds(base + c, _CHUNK)])

    return gather_kernel(emb, idx)


def _lrelu(x):
    return jnp.where(x >= 0, x, 0.01 * x)


def _dense_body(fc_ref, fl_ref, ve_ref, w1_ref, b1_ref, w2_ref, b2_ref,
                w3_ref, b3_ref, wloc_ref, bloc_ref, wcls_ref,
                cls_ref, cos_ref, vcomp_ref, vlc_ref, vlcm_ref):
    f32 = jnp.float32
    x = fc_ref[...]
    h = _lrelu(jnp.dot(x, w1_ref[...], preferred_element_type=f32) + b1_ref[...])
    h = _lrelu(jnp.dot(h, w2_ref[...], preferred_element_type=f32) + b2_ref[...])
    v_comp = _lrelu(jnp.dot(h, w3_ref[...], preferred_element_type=f32) + b3_ref[...])
    v_loc = _lrelu(jnp.dot(fl_ref[...], wloc_ref[...], preferred_element_type=f32)
                   + bloc_ref[...])

    ve = ve_ref[...]
    vc16 = v_comp[:, :16]
    vc80 = v_comp[:, 16:]
    n1 = jnp.sqrt(jnp.sum(v_comp * v_comp, axis=1, keepdims=True))
    nloc2 = jnp.sum(v_loc * v_loc, axis=1, keepdims=True)
    dloc = jnp.sum(vc16 * v_loc, axis=1, keepdims=True)

    best = None
    chosen = None
    pieces = []
    for k in range(DICT_NUM):
        ve_k = ve[:, k * DICT_DIM:(k + 1) * DICT_DIM]
        pieces.append(v_loc)
        pieces.append(ve_k)
        dot_k = dloc + jnp.sum(vc80 * ve_k, axis=1, keepdims=True)
        n2_k = jnp.sqrt(nloc2 + jnp.sum(ve_k * ve_k, axis=1, keepdims=True))
        cos_k = dot_k / jnp.maximum(n1 * n2_k, 1e-8)
        if best is None:
            best = cos_k
            chosen = ve_k
        else:
            upd = cos_k > best
            best = jnp.where(upd, cos_k, best)
            chosen = jnp.where(upd, ve_k, chosen)

    vlc_ref[...] = jnp.concatenate(pieces, axis=1)
    cos_ref[...] = best
    vcomp_ref[...] = v_comp
    vlcm = jnp.concatenate([v_loc, chosen], axis=1)
    vlcm_ref[...] = vlcm
    v_diff = jnp.abs(vlcm - v_comp)
    cls_ref[...] = jnp.dot(v_diff, wcls_ref[...], preferred_element_type=f32)


def _dense(feat_comp, feat_loc, v_emb, W1, b1, W2, b2, W3, b3, Wloc, bloc, Wcls):
    b = feat_comp.shape[0]
    bb = 1024
    grid = (b // bb,)
    f32 = jnp.float32

    def row_spec(cols):
        return pl.BlockSpec((bb, cols), lambda i: (i, 0))

    def full_spec(shape):
        return pl.BlockSpec(shape, lambda i: (0, 0))

    weights = [W1.T, b1.reshape(1, -1), W2.T, b2.reshape(1, -1),
               W3.T, b3.reshape(1, -1), Wloc.T, bloc.reshape(1, -1), Wcls.T]

    out = pl.pallas_call(
        _dense_body,
        grid=grid,
        in_specs=[
            row_spec(feat_comp.shape[1]),
            row_spec(feat_loc.shape[1]),
            row_spec(v_emb.shape[1]),
        ] + [full_spec(w.shape) for w in weights],
        out_specs=[
            row_spec(2),
            row_spec(1),
            row_spec(COMMON),
            row_spec(DICT_NUM * COMMON),
            row_spec(COMMON),
        ],
        out_shape=[
            jax.ShapeDtypeStruct((b, 2), f32),
            jax.ShapeDtypeStruct((b, 1), f32),
            jax.ShapeDtypeStruct((b, COMMON), f32),
            jax.ShapeDtypeStruct((b, DICT_NUM * COMMON), f32),
            jax.ShapeDtypeStruct((b, COMMON), f32),
        ],
    )(feat_comp, feat_loc, v_emb, *weights)
    return out


@jax.jit
def kernel(feat_comp, feat_loc, id_loc, W1, b1, W2, b2, W3, b3, emb, Wloc, bloc, Wcls):
    vocab = emb.shape[0]
    emb_p = jnp.concatenate(
        [emb, jnp.zeros((vocab, 96), jnp.float32)], axis=1)
    v_emb = _sc_gather(emb_p, id_loc.astype(jnp.int32))
    cls, cos, vcomp, vlc, vlcm = _dense(
        feat_comp, feat_loc, v_emb, W1, b1, W2, b2, W3, b3, Wloc, bloc, Wcls)
    b = feat_comp.shape[0]
    return (cls, cos, vcomp, vlc.reshape(b, DICT_NUM, COMMON), vlcm)


# speedup vs baseline: 2.3322x; 2.0691x over previous
import functools

import jax
import jax.numpy as jnp
from jax import lax
from jax.experimental import pallas as pl
from jax.experimental.pallas import tpu as pltpu
from jax.experimental.pallas import tpu_sc as plsc

DICT_NUM = 10
DICT_DIM = 80
COMMON = 96

_NC = 2
_NS = 16
_NW = _NC * _NS
_CHUNK = 64


def _sc_gather(emb, idx):
    vocab, d = emb.shape
    b = idx.shape[0]
    b_per_w = b // _NW
    mesh = plsc.VectorSubcoreMesh(core_axis_name="c", subcore_axis_name="s")

    @functools.partial(
        pl.kernel,
        mesh=mesh,
        out_type=jax.ShapeDtypeStruct((b, d), jnp.float32),
        scratch_types=[
            pltpu.VMEM((b_per_w,), jnp.int32),
            pltpu.VMEM((_CHUNK, d), jnp.float32),
            pltpu.SemaphoreType.DMA,
        ],
    )
    def gather_kernel(table_hbm, idx_hbm, out_hbm, idx_v, rows_v, sem):
        wid = lax.axis_index("s") * _NC + lax.axis_index("c")
        base = wid * b_per_w
        pltpu.sync_copy(idx_hbm.at[pl.ds(base, b_per_w)], idx_v)

        @pl.loop(0, b_per_w, step=_CHUNK)
        def _(c):
            pltpu.async_copy(
                table_hbm.at[idx_v.at[pl.ds(c, _CHUNK)]], rows_v, sem
            ).wait()
            pltpu.sync_copy(rows_v, out_hbm.at[pl.ds(base + c, _CHUNK)])

    return gather_kernel(emb, idx)


def _lrelu(x):
    return jnp.where(x >= 0, x, 0.01 * x)


def _pad_body(src_ref, dst_ref):
    dst_ref[:, :800] = src_ref[...]
    dst_ref[:, 800:] = jnp.zeros((src_ref.shape[0], 96), jnp.float32)


def _pad_table(emb):
    vocab = emb.shape[0]
    rb = 1000
    return pl.pallas_call(
        _pad_body,
        grid=(vocab // rb,),
        in_specs=[pl.BlockSpec((rb, 800), lambda i: (i, 0))],
        out_specs=pl.BlockSpec((rb, 896), lambda i: (i, 0)),
        out_shape=jax.ShapeDtypeStruct((vocab, 896), jnp.float32),
    )(emb)


def _dense_body(fc_ref, fl_ref, ve_ref, w1_ref, b1_ref, w2_ref, b2_ref,
                w3_ref, b3_ref, wloc_ref, bloc_ref, wcls_ref,
                cls_ref, cos_ref, vcomp_ref, vlc_ref, vlcm_ref):
    f32 = jnp.float32
    x = fc_ref[...]
    h = _lrelu(jnp.dot(x, w1_ref[...], preferred_element_type=f32) + b1_ref[...])
    h = _lrelu(jnp.dot(h, w2_ref[...], preferred_element_type=f32) + b2_ref[...])
    v_comp = _lrelu(jnp.dot(h, w3_ref[...], preferred_element_type=f32) + b3_ref[...])
    v_loc = _lrelu(jnp.dot(fl_ref[...], wloc_ref[...], preferred_element_type=f32)
                   + bloc_ref[...])

    ve = ve_ref[...]
    vc16 = v_comp[:, :16]
    vc80 = v_comp[:, 16:]
    n1 = jnp.sqrt(jnp.sum(v_comp * v_comp, axis=1, keepdims=True))
    nloc2 = jnp.sum(v_loc * v_loc, axis=1, keepdims=True)
    dloc = jnp.sum(vc16 * v_loc, axis=1, keepdims=True)

    best = None
    chosen = None
    pieces = []
    for k in range(DICT_NUM):
        ve_k = ve[:, k * DICT_DIM:(k + 1) * DICT_DIM]
        pieces.append(v_loc)
        pieces.append(ve_k)
        dot_k = dloc + jnp.sum(vc80 * ve_k, axis=1, keepdims=True)
        n2_k = jnp.sqrt(nloc2 + jnp.sum(ve_k * ve_k, axis=1, keepdims=True))
        cos_k = dot_k / jnp.maximum(n1 * n2_k, 1e-8)
        if best is None:
            best = cos_k
            chosen = ve_k
        else:
            upd = cos_k > best
            best = jnp.where(upd, cos_k, best)
            chosen = jnp.where(upd, ve_k, chosen)

    vlc_ref[...] = jnp.concatenate(pieces, axis=1)
    cos_ref[...] = best
    vcomp_ref[...] = v_comp
    vlcm = jnp.concatenate([v_loc, chosen], axis=1)
    vlcm_ref[...] = vlcm
    v_diff = jnp.abs(vlcm - v_comp)
    cls_ref[...] = jnp.dot(v_diff, wcls_ref[...], preferred_element_type=f32)


def _dense(feat_comp, feat_loc, v_emb, W1, b1, W2, b2, W3, b3, Wloc, bloc, Wcls):
    b = feat_comp.shape[0]
    bb = 1024
    grid = (b // bb,)
    f32 = jnp.float32

    def row_spec(cols):
        return pl.BlockSpec((bb, cols), lambda i: (i, 0))

    def full_spec(shape):
        return pl.BlockSpec(shape, lambda i: (0, 0))

    weights = [W1.T, b1.reshape(1, -1), W2.T, b2.reshape(1, -1),
               W3.T, b3.reshape(1, -1), Wloc.T, bloc.reshape(1, -1), Wcls.T]

    out = pl.pallas_call(
        _dense_body,
        grid=grid,
        in_specs=[
            row_spec(feat_comp.shape[1]),
            row_spec(feat_loc.shape[1]),
            row_spec(v_emb.shape[1]),
        ] + [full_spec(w.shape) for w in weights],
        out_specs=[
            row_spec(2),
            row_spec(1),
            row_spec(COMMON),
            row_spec(DICT_NUM * COMMON),
            row_spec(COMMON),
        ],
        out_shape=[
            jax.ShapeDtypeStruct((b, 2), f32),
            jax.ShapeDtypeStruct((b, 1), f32),
            jax.ShapeDtypeStruct((b, COMMON), f32),
            jax.ShapeDtypeStruct((b, DICT_NUM * COMMON), f32),
            jax.ShapeDtypeStruct((b, COMMON), f32),
        ],
    )(feat_comp, feat_loc, v_emb, *weights)
    return out


@jax.jit
def kernel(feat_comp, feat_loc, id_loc, W1, b1, W2, b2, W3, b3, emb, Wloc, bloc, Wcls):
    emb_p = _pad_table(emb)
    v_emb = _sc_gather(emb_p, id_loc.astype(jnp.int32))
    cls, cos, vcomp, vlc, vlcm = _dense(
        feat_comp, feat_loc, v_emb, W1, b1, W2, b2, W3, b3, Wloc, bloc, Wcls)
    b = feat_comp.shape[0]
    return (cls, cos, vcomp, vlc.reshape(b, DICT_NUM, COMMON), vlcm)


# speedup vs baseline: 2.6931x; 1.1548x over previous
import functools

import jax
import jax.numpy as jnp
from jax import lax
from jax.experimental import pallas as pl
from jax.experimental.pallas import tpu as pltpu
from jax.experimental.pallas import tpu_sc as plsc

DICT_NUM = 10
DICT_DIM = 80
COMMON = 96

_NC = 2
_NS = 16
_NW = _NC * _NS
_CHUNK = 64


def _sc_gather(emb, idx):
    vocab, d = emb.shape
    b = idx.shape[0]
    b_per_w = b // _NW
    mesh = plsc.VectorSubcoreMesh(core_axis_name="c", subcore_axis_name="s")

    @functools.partial(
        pl.kernel,
        mesh=mesh,
        out_type=jax.ShapeDtypeStruct((b, d), jnp.float32),
        scratch_types=[
            pltpu.VMEM((b_per_w,), jnp.int32),
            pltpu.VMEM((_CHUNK, d), jnp.float32),
            pltpu.SemaphoreType.DMA,
        ],
    )
    def gather_kernel(table_hbm, idx_hbm, out_hbm, idx_v, rows_v, sem):
        wid = lax.axis_index("s") * _NC + lax.axis_index("c")
        base = wid * b_per_w
        pltpu.sync_copy(idx_hbm.at[pl.ds(base, b_per_w)], idx_v)

        @pl.loop(0, b_per_w, step=_CHUNK)
        def _(c):
            pltpu.async_copy(
                table_hbm.at[idx_v.at[pl.ds(c, _CHUNK)]], rows_v, sem
            ).wait()
            pltpu.sync_copy(rows_v, out_hbm.at[pl.ds(base + c, _CHUNK)])

    return gather_kernel(emb, idx)


def _lrelu(x):
    return jnp.where(x >= 0, x, 0.01 * x)


def _pad_body(src_ref, dst_ref):
    dst_ref[:, :800] = src_ref[...]
    dst_ref[:, 800:] = jnp.zeros((src_ref.shape[0], 96), jnp.float32)


def _pad_table(emb):
    vocab = emb.shape[0]
    rb = 1000
    return pl.pallas_call(
        _pad_body,
        grid=(vocab // rb,),
        in_specs=[pl.BlockSpec((rb, 800), lambda i: (i, 0))],
        out_specs=pl.BlockSpec((rb, 896), lambda i: (i, 0)),
        out_shape=jax.ShapeDtypeStruct((vocab, 896), jnp.float32),
    )(emb)


def _dense_body(fc_ref, fl_ref, ve_ref, w1_ref, b1_ref, w2_ref, b2_ref,
                w3_ref, b3_ref, wloc_ref, bloc_ref, wcls_ref,
                cls_ref, cos_ref, vcomp_ref, vlc_ref, vlcm_ref):
    f32 = jnp.float32
    x = fc_ref[...]
    h = _lrelu(jnp.dot(x, w1_ref[...], preferred_element_type=f32) + b1_ref[...])
    h = _lrelu(jnp.dot(h, w2_ref[...], preferred_element_type=f32) + b2_ref[...])
    v_comp = _lrelu(jnp.dot(h, w3_ref[...], preferred_element_type=f32) + b3_ref[...])
    v_loc = _lrelu(jnp.dot(fl_ref[...], wloc_ref[...], preferred_element_type=f32)
                   + bloc_ref[...])

    ve = ve_ref[...]
    n1 = jnp.sqrt(jnp.sum(v_comp * v_comp, axis=1, keepdims=True))

    best = None
    vlcm = None
    pieces = []
    for k in range(DICT_NUM):
        ve_k = ve[:, k * DICT_DIM:(k + 1) * DICT_DIM]
        cat_k = jnp.concatenate([v_loc, ve_k], axis=1)
        pieces.append(cat_k)
        dot_k = jnp.sum(v_comp * cat_k, axis=1, keepdims=True)
        n2_k = jnp.sqrt(jnp.sum(cat_k * cat_k, axis=1, keepdims=True))
        cos_k = dot_k / jnp.maximum(n1 * n2_k, 1e-8)
        if best is None:
            best = cos_k
            vlcm = cat_k
        else:
            upd = cos_k > best
            best = jnp.where(upd, cos_k, best)
            vlcm = jnp.where(upd, cat_k, vlcm)

    vlc_ref[...] = jnp.concatenate(pieces, axis=1)
    cos_ref[...] = best
    vcomp_ref[...] = v_comp
    vlcm_ref[...] = vlcm
    v_diff = jnp.abs(vlcm - v_comp)
    cls_ref[...] = jnp.dot(v_diff, wcls_ref[...], preferred_element_type=f32)


def _dense(feat_comp, feat_loc, v_emb, W1, b1, W2, b2, W3, b3, Wloc, bloc, Wcls):
    b = feat_comp.shape[0]
    bb = 1024
    grid = (b // bb,)
    f32 = jnp.float32

    def row_spec(cols):
        return pl.BlockSpec((bb, cols), lambda i: (i, 0))

    def full_spec(shape):
        return pl.BlockSpec(shape, lambda i: (0, 0))

    weights = [W1.T, b1.reshape(1, -1), W2.T, b2.reshape(1, -1),
               W3.T, b3.reshape(1, -1), Wloc.T, bloc.reshape(1, -1), Wcls.T]

    out = pl.pallas_call(
        _dense_body,
        grid=grid,
        in_specs=[
            row_spec(feat_comp.shape[1]),
            row_spec(feat_loc.shape[1]),
            row_spec(v_emb.shape[1]),
        ] + [full_spec(w.shape) for w in weights],
        out_specs=[
            row_spec(2),
            row_spec(1),
            row_spec(COMMON),
            row_spec(DICT_NUM * COMMON),
            row_spec(COMMON),
        ],
        out_shape=[
            jax.ShapeDtypeStruct((b, 2), f32),
            jax.ShapeDtypeStruct((b, 1), f32),
            jax.ShapeDtypeStruct((b, COMMON), f32),
            jax.ShapeDtypeStruct((b, DICT_NUM * COMMON), f32),
            jax.ShapeDtypeStruct((b, COMMON), f32),
        ],
    )(feat_comp, feat_loc, v_emb, *weights)
    return out


@jax.jit
def kernel(feat_comp, feat_loc, id_loc, W1, b1, W2, b2, W3, b3, emb, Wloc, bloc, Wcls):
    emb_p = _pad_table(emb)
    v_emb = _sc_gather(emb_p, id_loc.astype(jnp.int32))
    cls, cos, vcomp, vlc, vlcm = _dense(
        feat_comp, feat_loc, v_emb, W1, b1, W2, b2, W3, b3, Wloc, bloc, Wcls)
    b = feat_comp.shape[0]
    return (cls, cos, vcomp, vlc.reshape(b, DICT_NUM, COMMON), vlcm)
